# baseline (speedup 1.0000x reference)
import jax
import jax.numpy as jnp
from jax import lax
from jax.experimental import pallas as pl
from jax.experimental.pallas import tpu as pltpu
from jax.experimental.pallas import tpu_sc as plsc

_B, _S, _K = 32, 256, 8192
_R = _B * _S
_RB = 64
_CW = 2048

_SC_ROWS = 2432
_R0 = _R - _SC_ROWS
_NW = 32


def _rotl(x, d):
    return lax.shift_left(x, jnp.int32(d)) | lax.shift_right_logical(
        x, jnp.int32(32 - d)
    )


_KS0 = 0
_KS1 = 42
_KS2 = _KS0 ^ _KS1 ^ 0x1BD11BDA
_KS = (_KS0, _KS1, _KS2)
_ROTS = ((13, 15, 26, 6), (17, 29, 16, 24))


def _threefry_bits(x1):
    x0 = x1
    x1 = x0 ^ _rotl(x1, 13)
    for r in (15, 26, 6):
        x0 = x0 + x1
        x1 = _rotl(x1, r)
        x1 = x0 ^ x1
    x0 = x0 + jnp.int32(_KS[1])
    x1 = x1 + jnp.int32(_KS[2] + 1)
    for j in range(2, 6):
        for r in _ROTS[(j - 1) % 2]:
            x0 = x0 + x1
            x1 = _rotl(x1, r)
            x1 = x0 ^ x1
        x0 = x0 + jnp.int32(_KS[j % 3])
        x1 = x1 + jnp.int32((_KS[(j + 1) % 3] + j) % (1 << 32))
    return x0 ^ x1


def _gumbel_from_bits(bits):
    fb = lax.shift_right_logical(bits, jnp.int32(9)) | jnp.int32(0x3F800000)
    u = lax.bitcast_convert_type(fb, jnp.float32) - jnp.float32(1.0)
    return -jnp.log(-jnp.log(u))


def _main_body(x_ref, o_ref):
    i = pl.program_id(0)
    base = i * _RB * _K
    rr = lax.broadcasted_iota(jnp.int32, (_RB, _CW), 0)
    kk = lax.broadcasted_iota(jnp.int32, (_RB, _CW), 1)
    p0 = base + rr * _K + kk
    m = jnp.full((_RB, 1), -jnp.inf, jnp.float32)
    idx = jnp.zeros((_RB, 1), jnp.int32)
    for c in range(_K // _CW):
        bits = _threefry_bits(p0 + jnp.int32(c * _CW + 42))
        g = _gumbel_from_bits(bits)
        z = x_ref[:, c * _CW:(c + 1) * _CW] + g
        mc = jnp.max(z, axis=-1, keepdims=True)
        cand = jnp.where(z == mc, kk, jnp.int32(_CW))
        ic = jnp.min(cand, axis=-1, keepdims=True) + jnp.int32(c * _CW)
        better = mc > m
        m = jnp.where(better, mc, m)
        idx = jnp.where(better, ic, idx)
    for c in range(_K // _CW):
        o_ref[:, c * _CW:(c + 1) * _CW] = jnp.where(
            kk == idx - jnp.int32(c * _CW), jnp.float32(1.0), jnp.float32(0.0)
        )


def _tail_body(x_ref, b_ref, prev_ref, o_ref):
    del prev_ref
    kk = lax.broadcasted_iota(jnp.int32, (_RB, _CW), 1)
    m = jnp.full((_RB, 1), -jnp.inf, jnp.float32)
    idx = jnp.zeros((_RB, 1), jnp.int32)
    for c in range(_K // _CW):
        g = _gumbel_from_bits(b_ref[:, c * _CW:(c + 1) * _CW])
        z = x_ref[:, c * _CW:(c + 1) * _CW] + g
        mc = jnp.max(z, axis=-1, keepdims=True)
        cand = jnp.where(z == mc, kk, jnp.int32(_CW))
        ic = jnp.min(cand, axis=-1, keepdims=True) + jnp.int32(c * _CW)
        better = mc > m
        m = jnp.where(better, mc, m)
        idx = jnp.where(better, ic, idx)
    for c in range(_K // _CW):
        o_ref[:, c * _CW:(c + 1) * _CW] = jnp.where(
            kk == idx - jnp.int32(c * _CW), jnp.float32(1.0), jnp.float32(0.0)
        )


_RPW = _SC_ROWS // _NW


def _sc_body(bits_hbm, buf):
    wid = lax.axis_index("s") * 2 + lax.axis_index("c")
    iota = lax.iota(jnp.int32, 16)

    def outer(g, carry):
        row = wid * _RPW + g
        gbase = (_R0 + row) * _K

        def inner(i, c2):
            for u in range(8):
                off = i * 128 + u * 16
                bits = _threefry_bits(iota + (gbase + off + 42))
                buf[pl.ds(off, 16)] = bits
            return c2

        lax.fori_loop(0, _K // 128, inner, 0)
        pltpu.sync_copy(buf, bits_hbm.at[row])
        return carry

    lax.fori_loop(0, _RPW, outer, 0)


_sc_bits = pl.kernel(
    _sc_body,
    out_type=jax.ShapeDtypeStruct((_SC_ROWS, _K), jnp.int32),
    mesh=plsc.VectorSubcoreMesh(core_axis_name="c", subcore_axis_name="s"),
    scratch_types=[pltpu.VMEM((_K,), jnp.int32)],
)


@jax.jit
def kernel(logits):
    x = logits.reshape(_R, _K)
    bits = _sc_bits()
    out_main = pl.pallas_call(
        _main_body,
        grid=(_R0 // _RB,),
        in_specs=[pl.BlockSpec((_RB, _K), lambda i: (i, 0))],
        out_specs=pl.BlockSpec((_RB, _K), lambda i: (i, 0)),
        out_shape=jax.ShapeDtypeStruct((_R, _K), jnp.float32),
        compiler_params=pltpu.CompilerParams(
            dimension_semantics=("parallel",)
        ),
    )(x)
    off = _R0 // _RB
    out = pl.pallas_call(
        _tail_body,
        grid=(_SC_ROWS // _RB,),
        in_specs=[
            pl.BlockSpec((_RB, _K), lambda i: (i + off, 0)),
            pl.BlockSpec((_RB, _K), lambda i: (i, 0)),
            pl.BlockSpec(memory_space=pl.ANY),
        ],
        out_specs=pl.BlockSpec((_RB, _K), lambda i: (i + off, 0)),
        out_shape=jax.ShapeDtypeStruct((_R, _K), jnp.float32),
        input_output_aliases={2: 0},
        compiler_params=pltpu.CompilerParams(
            dimension_semantics=("parallel",)
        ),
    )(x, bits, out_main)
    return out.reshape(_B, _S, _K)


# speedup vs baseline: 1.0168x; 1.0168x over previous
import jax
import jax.numpy as jnp
from jax import lax
from jax.experimental import pallas as pl
from jax.experimental.pallas import tpu as pltpu
from jax.experimental.pallas import tpu_sc as plsc

_B, _S, _K = 32, 256, 8192
_R = _B * _S
_RB = 64
_CW = 2048

_SC_ROWS = 2368
_R0 = _R - _SC_ROWS
_NW = 32


def _rotl(x, d):
    return lax.shift_left(x, jnp.int32(d)) | lax.shift_right_logical(
        x, jnp.int32(32 - d)
    )


_KS0 = 0
_KS1 = 42
_KS2 = _KS0 ^ _KS1 ^ 0x1BD11BDA
_KS = (_KS0, _KS1, _KS2)
_ROTS = ((13, 15, 26, 6), (17, 29, 16, 24))


def _threefry_bits(x1):
    x0 = x1
    x1 = x0 ^ _rotl(x1, 13)
    for r in (15, 26, 6):
        x0 = x0 + x1
        x1 = _rotl(x1, r)
        x1 = x0 ^ x1
    x0 = x0 + jnp.int32(_KS[1])
    x1 = x1 + jnp.int32(_KS[2] + 1)
    for j in range(2, 6):
        for r in _ROTS[(j - 1) % 2]:
            x0 = x0 + x1
            x1 = _rotl(x1, r)
            x1 = x0 ^ x1
        x0 = x0 + jnp.int32(_KS[j % 3])
        x1 = x1 + jnp.int32((_KS[(j + 1) % 3] + j) % (1 << 32))
    return x0 ^ x1


def _gumbel_from_bits(bits):
    fb = lax.shift_right_logical(bits, jnp.int32(9)) | jnp.int32(0x3F800000)
    u = lax.bitcast_convert_type(fb, jnp.float32) - jnp.float32(1.0)
    return -jnp.log(-jnp.log(u))


def _main_body(x_ref, o_ref):
    i = pl.program_id(0)
    base = i * _RB * _K
    rr = lax.broadcasted_iota(jnp.int32, (_RB, _CW), 0)
    kk = lax.broadcasted_iota(jnp.int32, (_RB, _CW), 1)
    p0 = base + rr * _K + kk
    m = jnp.full((_RB, 1), -jnp.inf, jnp.float32)
    idx = jnp.zeros((_RB, 1), jnp.int32)
    for c in range(_K // _CW):
        bits = _threefry_bits(p0 + jnp.int32(c * _CW + 42))
        g = _gumbel_from_bits(bits)
        z = x_ref[:, c * _CW:(c + 1) * _CW] + g
        mc = jnp.max(z, axis=-1, keepdims=True)
        cand = jnp.where(z == mc, kk, jnp.int32(_CW))
        ic = jnp.min(cand, axis=-1, keepdims=True) + jnp.int32(c * _CW)
        better = mc > m
        m = jnp.where(better, mc, m)
        idx = jnp.where(better, ic, idx)
    for c in range(_K // _CW):
        o_ref[:, c * _CW:(c + 1) * _CW] = jnp.where(
            kk == idx - jnp.int32(c * _CW), jnp.float32(1.0), jnp.float32(0.0)
        )


def _tail_body(x_ref, b_ref, prev_ref, o_ref):
    del prev_ref
    kk = lax.broadcasted_iota(jnp.int32, (_RB, _CW), 1)
    m = jnp.full((_RB, 1), -jnp.inf, jnp.float32)
    idx = jnp.zeros((_RB, 1), jnp.int32)
    for c in range(_K // _CW):
        g = _gumbel_from_bits(b_ref[:, c * _CW:(c + 1) * _CW])
        z = x_ref[:, c * _CW:(c + 1) * _CW] + g
        mc = jnp.max(z, axis=-1, keepdims=True)
        cand = jnp.where(z == mc, kk, jnp.int32(_CW))
        ic = jnp.min(cand, axis=-1, keepdims=True) + jnp.int32(c * _CW)
        better = mc > m
        m = jnp.where(better, mc, m)
        idx = jnp.where(better, ic, idx)
    for c in range(_K // _CW):
        o_ref[:, c * _CW:(c + 1) * _CW] = jnp.where(
            kk == idx - jnp.int32(c * _CW), jnp.float32(1.0), jnp.float32(0.0)
        )


_RPW = _SC_ROWS // _NW


def _sc_body(bits_hbm, buf):
    wid = lax.axis_index("s") * 2 + lax.axis_index("c")
    iota = lax.iota(jnp.int32, 16)

    def outer(g, carry):
        row = wid * _RPW + g
        gbase = (_R0 + row) * _K

        def inner(i, c2):
            for u in range(8):
                off = i * 128 + u * 16
                bits = _threefry_bits(iota + (gbase + off + 42))
                buf[pl.ds(off, 16)] = bits
            return c2

        lax.fori_loop(0, _K // 128, inner, 0)
        pltpu.sync_copy(buf, bits_hbm.at[row])
        return carry

    lax.fori_loop(0, _RPW, outer, 0)


_sc_bits = pl.kernel(
    _sc_body,
    out_type=jax.ShapeDtypeStruct((_SC_ROWS, _K), jnp.int32),
    mesh=plsc.VectorSubcoreMesh(core_axis_name="c", subcore_axis_name="s"),
    scratch_types=[pltpu.VMEM((_K,), jnp.int32)],
)


@jax.jit
def kernel(logits):
    x = logits.reshape(_R, _K)
    bits = _sc_bits()
    out_main = pl.pallas_call(
        _main_body,
        grid=(_R0 // _RB,),
        in_specs=[pl.BlockSpec((_RB, _K), lambda i: (i, 0))],
        out_specs=pl.BlockSpec((_RB, _K), lambda i: (i, 0)),
        out_shape=jax.ShapeDtypeStruct((_R, _K), jnp.float32),
        compiler_params=pltpu.CompilerParams(
            dimension_semantics=("parallel",)
        ),
    )(x)
    off = _R0 // _RB
    out = pl.pallas_call(
        _tail_body,
        grid=(_SC_ROWS // _RB,),
        in_specs=[
            pl.BlockSpec((_RB, _K), lambda i: (i + off, 0)),
            pl.BlockSpec((_RB, _K), lambda i: (i, 0)),
            pl.BlockSpec(memory_space=pl.ANY),
        ],
        out_specs=pl.BlockSpec((_RB, _K), lambda i: (i + off, 0)),
        out_shape=jax.ShapeDtypeStruct((_R, _K), jnp.float32),
        input_output_aliases={2: 0},
        compiler_params=pltpu.CompilerParams(
            dimension_semantics=("parallel",)
        ),
    )(x, bits, out_main)
    return out.reshape(_B, _S, _K)
